# baseline (speedup 1.0000x reference)
import functools

import jax
import jax.numpy as jnp
from jax.experimental import pallas as pl
from jax.experimental.pallas import tpu as pltpu

_G = 512


def _fps_body(x_ref, y_ref, z_ref, idx_ref, cx_ref, cy_ref, cz_ref, dists_ref):
    x = x_ref[...]
    y = y_ref[...]
    z = z_ref[...]
    B, N = x.shape
    iota = jax.lax.broadcasted_iota(jnp.int32, (B, N), 1)

    giota = jax.lax.broadcasted_iota(jnp.int32, (B, _G), 1)

    lx = x[:, 0:1]
    ly = y[:, 0:1]
    lz = z[:, 0:1]
    idx_acc = jnp.zeros((B, _G), jnp.int32)
    cx_acc = jnp.broadcast_to(lx, (B, _G))
    cy_acc = jnp.broadcast_to(ly, (B, _G))
    cz_acc = jnp.broadcast_to(lz, (B, _G))
    dists_ref[...] = jnp.full((B, N), jnp.inf, jnp.float32)

    def body(i, carry):
        lx, ly, lz, idx_acc, cx_acc, cy_acc, cz_acc = carry
        dx = x - lx
        dy = y - ly
        dz = z - lz
        d = (dx * dx + dy * dy) + dz * dz
        dists = jnp.minimum(dists_ref[...], d)
        dists_ref[...] = dists
        nxt = jnp.argmax(dists, axis=1).astype(jnp.int32)[:, None]
        onehot = iota == nxt
        zero = jnp.zeros((), jnp.float32)
        nlx = jnp.sum(jnp.where(onehot, x, zero), axis=1, keepdims=True)
        nly = jnp.sum(jnp.where(onehot, y, zero), axis=1, keepdims=True)
        nlz = jnp.sum(jnp.where(onehot, z, zero), axis=1, keepdims=True)
        sel = giota == i
        idx_acc = jnp.where(sel, nxt, idx_acc)
        cx_acc = jnp.where(sel, nlx, cx_acc)
        cy_acc = jnp.where(sel, nly, cy_acc)
        cz_acc = jnp.where(sel, nlz, cz_acc)
        return nlx, nly, nlz, idx_acc, cx_acc, cy_acc, cz_acc

    _, _, _, idx_acc, cx_acc, cy_acc, cz_acc = jax.lax.fori_loop(
        1, _G, body, (lx, ly, lz, idx_acc, cx_acc, cy_acc, cz_acc)
    )
    idx_ref[...] = idx_acc
    cx_ref[...] = cx_acc
    cy_ref[...] = cy_acc
    cz_ref[...] = cz_acc


def _group_body(xyzp_ref, ctr_ref, chi_ref, cmid_ref, clo_ref, feat_ref,
                gf_ref, nn_ref):
    B = xyzp_ref.shape[0]
    C = xyzp_ref.shape[1]
    for b in range(B):
        pts = xyzp_ref[b]
        cb = ctr_ref[b]
        dots = jax.lax.dot_general(
            pts, cb, (((1,), (0,)), ((), ())),
            preferred_element_type=jnp.float32,
        )
        q2 = jnp.sum(pts * pts, axis=1, keepdims=True)
        k2 = jnp.sum(cb * cb, axis=0, keepdims=True)
        d2 = (q2 + k2) - 2.0 * dots
        giota = jax.lax.broadcasted_iota(jnp.int32, (C, _G), 1)
        nnb = jnp.argmin(d2, axis=1).astype(jnp.int32)[:, None]
        onehot = (giota == nnb).astype(jnp.bfloat16)
        def oh_dot(cref):
            return jax.lax.dot_general(
                onehot, cref[b], (((1,), (0,)), ((), ())),
                preferred_element_type=jnp.float32,
            )
        csel = (oh_dot(chi_ref) + oh_dot(cmid_ref)) + oh_dot(clo_ref)
        nbr = pts - csel
        s = jnp.sum(nbr * nbr, axis=1, keepdims=True)
        dist = jnp.sqrt(s + 1e-16)
        nrm = nbr / jnp.maximum(dist, 1e-8)
        gf_ref[b, :, 0:3] = nrm[:, 0:3]
        gf_ref[b, :, 3:4] = dist
        gf_ref[b, :, 4:68] = feat_ref[b]
        nn_ref[b, :] = nnb[:, 0]


@jax.jit
def kernel(xyz, features):
    B, N, _ = xyz.shape
    F = features.shape[-1]
    xt = jnp.transpose(xyz, (0, 2, 1))
    x, y, z = xt[:, 0], xt[:, 1], xt[:, 2]

    idx, cx, cy, cz = pl.pallas_call(
        _fps_body,
        out_shape=[
            jax.ShapeDtypeStruct((B, _G), jnp.int32),
            jax.ShapeDtypeStruct((B, _G), jnp.float32),
            jax.ShapeDtypeStruct((B, _G), jnp.float32),
            jax.ShapeDtypeStruct((B, _G), jnp.float32),
        ],
        scratch_shapes=[pltpu.VMEM((B, N), jnp.float32)],
    )(x, y, z)

    centers = jnp.stack([cx, cy, cz], axis=-1)
    xyzp = jnp.pad(xyz, ((0, 0), (0, 0), (0, 5)))
    ctr = jnp.pad(jnp.stack([cx, cy, cz], axis=1),
                  ((0, 0), (0, 5), (0, 0)))
    ctrt = jnp.pad(centers, ((0, 0), (0, 0), (0, 5)))

    def trunc_bf16(v):
        return jax.lax.bitcast_convert_type(
            jax.lax.bitcast_convert_type(v, jnp.uint32) & jnp.uint32(0xFFFF0000),
            jnp.float32)
    chi_f = trunc_bf16(ctrt)
    cmid_f = trunc_bf16(ctrt - chi_f)
    clo_f = (ctrt - chi_f) - cmid_f
    chi = chi_f.astype(jnp.bfloat16)
    cmid = cmid_f.astype(jnp.bfloat16)
    clo = clo_f.astype(jnp.bfloat16)

    CH = 8
    CS = N // CH
    gf, nn = pl.pallas_call(
        _group_body,
        grid=(CH,),
        in_specs=[
            pl.BlockSpec((B, CS, 8), lambda c: (0, c, 0)),
            pl.BlockSpec((B, 8, _G), lambda c: (0, 0, 0)),
            pl.BlockSpec((B, _G, 8), lambda c: (0, 0, 0)),
            pl.BlockSpec((B, _G, 8), lambda c: (0, 0, 0)),
            pl.BlockSpec((B, _G, 8), lambda c: (0, 0, 0)),
            pl.BlockSpec((B, CS, F), lambda c: (0, c, 0)),
        ],
        out_specs=[
            pl.BlockSpec((B, CS, 4 + F), lambda c: (0, c, 0)),
            pl.BlockSpec((B, CS), lambda c: (0, c)),
        ],
        out_shape=[
            jax.ShapeDtypeStruct((B, N, 4 + F), jnp.float32),
            jax.ShapeDtypeStruct((B, N), jnp.int32),
        ],
    )(xyzp, ctr, chi, cmid, clo, features)

    return gf, centers, nn


# speedup vs baseline: 1.0913x; 1.0913x over previous
import functools

import jax
import jax.numpy as jnp
from jax.experimental import pallas as pl
from jax.experimental.pallas import tpu as pltpu

_G = 512


def _fps_body(x_ref, y_ref, z_ref, idx_ref, cx_ref, cy_ref, cz_ref, dists_ref):
    x = x_ref[...]
    y = y_ref[...]
    z = z_ref[...]
    B, N = x.shape
    iota = jax.lax.broadcasted_iota(jnp.int32, (B, N), 1)

    giota = jax.lax.broadcasted_iota(jnp.int32, (B, _G), 1)

    lx = x[:, 0:1]
    ly = y[:, 0:1]
    lz = z[:, 0:1]
    idx_acc = jnp.zeros((B, _G), jnp.int32)
    cx_acc = jnp.broadcast_to(lx, (B, _G))
    cy_acc = jnp.broadcast_to(ly, (B, _G))
    cz_acc = jnp.broadcast_to(lz, (B, _G))
    dists_ref[...] = jnp.full((B, N), jnp.inf, jnp.float32)

    def body(i, carry):
        lx, ly, lz, idx_acc, cx_acc, cy_acc, cz_acc = carry
        dx = x - lx
        dy = y - ly
        dz = z - lz
        d = (dx * dx + dy * dy) + dz * dz
        dists = jnp.minimum(dists_ref[...], d)
        dists_ref[...] = dists
        nxt = jnp.argmax(dists, axis=1).astype(jnp.int32)[:, None]
        onehot = iota == nxt
        zero = jnp.zeros((), jnp.float32)
        nlx = jnp.sum(jnp.where(onehot, x, zero), axis=1, keepdims=True)
        nly = jnp.sum(jnp.where(onehot, y, zero), axis=1, keepdims=True)
        nlz = jnp.sum(jnp.where(onehot, z, zero), axis=1, keepdims=True)
        sel = giota == i
        idx_acc = jnp.where(sel, nxt, idx_acc)
        cx_acc = jnp.where(sel, nlx, cx_acc)
        cy_acc = jnp.where(sel, nly, cy_acc)
        cz_acc = jnp.where(sel, nlz, cz_acc)
        return nlx, nly, nlz, idx_acc, cx_acc, cy_acc, cz_acc

    _, _, _, idx_acc, cx_acc, cy_acc, cz_acc = jax.lax.fori_loop(
        1, _G, body, (lx, ly, lz, idx_acc, cx_acc, cy_acc, cz_acc)
    )
    idx_ref[...] = idx_acc
    cx_ref[...] = cx_acc
    cy_ref[...] = cy_acc
    cz_ref[...] = cz_acc


def _group_body(xyzp_ref, ctr_ref, chi_ref, cmid_ref, clo_ref,
                gf_ref, nn_ref):
    B = xyzp_ref.shape[0]
    C = xyzp_ref.shape[1]
    for b in range(B):
        pts = xyzp_ref[b]
        cb = ctr_ref[b]
        dots = jax.lax.dot_general(
            pts, cb, (((1,), (0,)), ((), ())),
            preferred_element_type=jnp.float32,
        )
        q2 = jnp.sum(pts * pts, axis=1, keepdims=True)
        k2 = jnp.sum(cb * cb, axis=0, keepdims=True)
        d2 = (q2 + k2) - 2.0 * dots
        giota = jax.lax.broadcasted_iota(jnp.int32, (C, _G), 1)
        nnb = jnp.argmin(d2, axis=1).astype(jnp.int32)[:, None]
        onehot = (giota == nnb).astype(jnp.bfloat16)
        def oh_dot(cref):
            return jax.lax.dot_general(
                onehot, cref[b], (((1,), (0,)), ((), ())),
                preferred_element_type=jnp.float32,
            )
        csel = (oh_dot(chi_ref) + oh_dot(cmid_ref)) + oh_dot(clo_ref)
        nbr = pts - csel
        s = jnp.sum(nbr * nbr, axis=1, keepdims=True)
        dist = jnp.sqrt(s + 1e-16)
        nrm = nbr / jnp.maximum(dist, 1e-8)
        gf_ref[b, :, 0:3] = nrm[:, 0:3]
        gf_ref[b, :, 3:4] = dist
        nn_ref[b, :] = nnb[:, 0]


@jax.jit
def kernel(xyz, features):
    B, N, _ = xyz.shape
    F = features.shape[-1]
    xt = jnp.transpose(xyz, (0, 2, 1))
    x, y, z = xt[:, 0], xt[:, 1], xt[:, 2]

    idx, cx, cy, cz = pl.pallas_call(
        _fps_body,
        out_shape=[
            jax.ShapeDtypeStruct((B, _G), jnp.int32),
            jax.ShapeDtypeStruct((B, _G), jnp.float32),
            jax.ShapeDtypeStruct((B, _G), jnp.float32),
            jax.ShapeDtypeStruct((B, _G), jnp.float32),
        ],
        scratch_shapes=[pltpu.VMEM((B, N), jnp.float32)],
    )(x, y, z)

    centers = jnp.stack([cx, cy, cz], axis=-1)
    xyzp = jnp.pad(xyz, ((0, 0), (0, 0), (0, 5)))
    ctr = jnp.pad(jnp.stack([cx, cy, cz], axis=1),
                  ((0, 0), (0, 5), (0, 0)))
    ctrt = jnp.pad(centers, ((0, 0), (0, 0), (0, 5)))

    def trunc_bf16(v):
        return jax.lax.bitcast_convert_type(
            jax.lax.bitcast_convert_type(v, jnp.uint32) & jnp.uint32(0xFFFF0000),
            jnp.float32)
    chi_f = trunc_bf16(ctrt)
    cmid_f = trunc_bf16(ctrt - chi_f)
    clo_f = (ctrt - chi_f) - cmid_f
    chi = chi_f.astype(jnp.bfloat16)
    cmid = cmid_f.astype(jnp.bfloat16)
    clo = clo_f.astype(jnp.bfloat16)

    CH = 8
    CS = N // CH
    gfa, nn = pl.pallas_call(
        _group_body,
        grid=(CH,),
        in_specs=[
            pl.BlockSpec((B, CS, 8), lambda c: (0, c, 0)),
            pl.BlockSpec((B, 8, _G), lambda c: (0, 0, 0)),
            pl.BlockSpec((B, _G, 8), lambda c: (0, 0, 0)),
            pl.BlockSpec((B, _G, 8), lambda c: (0, 0, 0)),
            pl.BlockSpec((B, _G, 8), lambda c: (0, 0, 0)),
        ],
        out_specs=[
            pl.BlockSpec((B, CS, 4), lambda c: (0, c, 0)),
            pl.BlockSpec((B, CS), lambda c: (0, c)),
        ],
        out_shape=[
            jax.ShapeDtypeStruct((B, N, 4), jnp.float32),
            jax.ShapeDtypeStruct((B, N), jnp.int32),
        ],
    )(xyzp, ctr, chi, cmid, clo)

    gf = jnp.concatenate([gfa, features], axis=-1)
    return gf, centers, nn


# speedup vs baseline: 1.0935x; 1.0021x over previous
import functools

import jax
import jax.numpy as jnp
from jax.experimental import pallas as pl
from jax.experimental.pallas import tpu as pltpu

_G = 512


def _fps_body(x_ref, y_ref, z_ref, idx_ref, cx_ref, cy_ref, cz_ref, dists_ref):
    x = x_ref[...]
    y = y_ref[...]
    z = z_ref[...]
    B, N = x.shape
    iota = jax.lax.broadcasted_iota(jnp.int32, (B, N), 1)

    giota = jax.lax.broadcasted_iota(jnp.int32, (B, _G), 1)

    lx = x[:, 0:1]
    ly = y[:, 0:1]
    lz = z[:, 0:1]
    idx_ref[...] = jnp.zeros((B, _G), jnp.int32)
    cx_ref[...] = jnp.broadcast_to(lx, (B, _G))
    cy_ref[...] = jnp.broadcast_to(ly, (B, _G))
    cz_ref[...] = jnp.broadcast_to(lz, (B, _G))
    dists_ref[...] = jnp.full((B, N), jnp.inf, jnp.float32)

    def body(i, carry):
        lx, ly, lz = carry
        dx = x - lx
        dy = y - ly
        dz = z - lz
        d = (dx * dx + dy * dy) + dz * dz
        dists = jnp.minimum(dists_ref[...], d)
        dists_ref[...] = dists
        nxt = jnp.argmax(dists, axis=1).astype(jnp.int32)[:, None]
        onehot = iota == nxt
        zero = jnp.zeros((), jnp.float32)
        nlx = jnp.sum(jnp.where(onehot, x, zero), axis=1, keepdims=True)
        nly = jnp.sum(jnp.where(onehot, y, zero), axis=1, keepdims=True)
        nlz = jnp.sum(jnp.where(onehot, z, zero), axis=1, keepdims=True)
        sel = giota == i
        idx_ref[...] = jnp.where(sel, nxt, idx_ref[...])
        cx_ref[...] = jnp.where(sel, nlx, cx_ref[...])
        cy_ref[...] = jnp.where(sel, nly, cy_ref[...])
        cz_ref[...] = jnp.where(sel, nlz, cz_ref[...])
        return nlx, nly, nlz

    jax.lax.fori_loop(1, _G, body, (lx, ly, lz))


def _group_body(xyzp_ref, ctr_ref, chi_ref, cmid_ref, clo_ref,
                gf_ref, nn_ref):
    B = xyzp_ref.shape[0]
    C = xyzp_ref.shape[1]
    for b in range(B):
        pts = xyzp_ref[b]
        cb = ctr_ref[b]
        dots = jax.lax.dot_general(
            pts, cb, (((1,), (0,)), ((), ())),
            preferred_element_type=jnp.float32,
        )
        q2 = jnp.sum(pts * pts, axis=1, keepdims=True)
        k2 = jnp.sum(cb * cb, axis=0, keepdims=True)
        d2 = (q2 + k2) - 2.0 * dots
        giota = jax.lax.broadcasted_iota(jnp.int32, (C, _G), 1)
        nnb = jnp.argmin(d2, axis=1).astype(jnp.int32)[:, None]
        onehot = (giota == nnb).astype(jnp.bfloat16)
        def oh_dot(cref):
            return jax.lax.dot_general(
                onehot, cref[b], (((1,), (0,)), ((), ())),
                preferred_element_type=jnp.float32,
            )
        csel = (oh_dot(chi_ref) + oh_dot(cmid_ref)) + oh_dot(clo_ref)
        nbr = pts - csel
        s = jnp.sum(nbr * nbr, axis=1, keepdims=True)
        dist = jnp.sqrt(s + 1e-16)
        nrm = nbr / jnp.maximum(dist, 1e-8)
        gf_ref[b, :, 0:3] = nrm[:, 0:3]
        gf_ref[b, :, 3:4] = dist
        nn_ref[b, :] = nnb[:, 0]


@jax.jit
def kernel(xyz, features):
    B, N, _ = xyz.shape
    F = features.shape[-1]
    xt = jnp.transpose(xyz, (0, 2, 1))
    x, y, z = xt[:, 0], xt[:, 1], xt[:, 2]

    idx, cx, cy, cz = pl.pallas_call(
        _fps_body,
        out_shape=[
            jax.ShapeDtypeStruct((B, _G), jnp.int32),
            jax.ShapeDtypeStruct((B, _G), jnp.float32),
            jax.ShapeDtypeStruct((B, _G), jnp.float32),
            jax.ShapeDtypeStruct((B, _G), jnp.float32),
        ],
        scratch_shapes=[pltpu.VMEM((B, N), jnp.float32)],
    )(x, y, z)

    centers = jnp.stack([cx, cy, cz], axis=-1)
    xyzp = jnp.pad(xyz, ((0, 0), (0, 0), (0, 5)))
    ctr = jnp.pad(jnp.stack([cx, cy, cz], axis=1),
                  ((0, 0), (0, 5), (0, 0)))
    ctrt = jnp.pad(centers, ((0, 0), (0, 0), (0, 5)))

    def trunc_bf16(v):
        return jax.lax.bitcast_convert_type(
            jax.lax.bitcast_convert_type(v, jnp.uint32) & jnp.uint32(0xFFFF0000),
            jnp.float32)
    chi_f = trunc_bf16(ctrt)
    cmid_f = trunc_bf16(ctrt - chi_f)
    clo_f = (ctrt - chi_f) - cmid_f
    chi = chi_f.astype(jnp.bfloat16)
    cmid = cmid_f.astype(jnp.bfloat16)
    clo = clo_f.astype(jnp.bfloat16)

    CH = 8
    CS = N // CH
    gfa, nn = pl.pallas_call(
        _group_body,
        grid=(CH,),
        in_specs=[
            pl.BlockSpec((B, CS, 8), lambda c: (0, c, 0)),
            pl.BlockSpec((B, 8, _G), lambda c: (0, 0, 0)),
            pl.BlockSpec((B, _G, 8), lambda c: (0, 0, 0)),
            pl.BlockSpec((B, _G, 8), lambda c: (0, 0, 0)),
            pl.BlockSpec((B, _G, 8), lambda c: (0, 0, 0)),
        ],
        out_specs=[
            pl.BlockSpec((B, CS, 4), lambda c: (0, c, 0)),
            pl.BlockSpec((B, CS), lambda c: (0, c)),
        ],
        out_shape=[
            jax.ShapeDtypeStruct((B, N, 4), jnp.float32),
            jax.ShapeDtypeStruct((B, N), jnp.int32),
        ],
    )(xyzp, ctr, chi, cmid, clo)

    gf = jnp.concatenate([gfa, features], axis=-1)
    return gf, centers, nn


# speedup vs baseline: 1.1103x; 1.0153x over previous
import functools

import jax
import jax.numpy as jnp
from jax.experimental import pallas as pl
from jax.experimental.pallas import tpu as pltpu

_G = 512


def _fps_body(x_ref, y_ref, z_ref, idx_ref, cx_ref, cy_ref, cz_ref, dists_ref):
    x = x_ref[...]
    y = y_ref[...]
    z = z_ref[...]
    B, N = x.shape
    iota = jax.lax.broadcasted_iota(jnp.int32, (B, N), 1)

    giota = jax.lax.broadcasted_iota(jnp.int32, (B, _G), 1)

    lx = x[:, 0:1]
    ly = y[:, 0:1]
    lz = z[:, 0:1]
    idx_ref[...] = jnp.zeros((B, _G), jnp.int32)
    cx_ref[...] = jnp.broadcast_to(lx, (B, _G))
    cy_ref[...] = jnp.broadcast_to(ly, (B, _G))
    cz_ref[...] = jnp.broadcast_to(lz, (B, _G))
    dists_ref[...] = jnp.full((B, N), jnp.inf, jnp.float32)

    def body(i, carry):
        lx, ly, lz = carry
        dx = x - lx
        dy = y - ly
        dz = z - lz
        d = (dx * dx + dy * dy) + dz * dz
        dists = jnp.minimum(dists_ref[...], d)
        dists_ref[...] = dists

        t, ti, tx, ty, tz = dists, iota, x, y, z
        w = N
        while w > 128:
            h = w // 2
            a, b = t[:, :h], t[:, h:w]
            cond = a >= b
            t = jnp.where(cond, a, b)
            ti = jnp.where(cond, ti[:, :h], ti[:, h:w])
            tx = jnp.where(cond, tx[:, :h], tx[:, h:w])
            ty = jnp.where(cond, ty[:, :h], ty[:, h:w])
            tz = jnp.where(cond, tz[:, :h], tz[:, h:w])
            w = h
        maxv = jnp.max(t, axis=1, keepdims=True)
        msk = t == maxv
        nxt = jnp.min(jnp.where(msk, ti, jnp.int32(N)), axis=1, keepdims=True)
        sel1 = msk & (ti == nxt)
        zero = jnp.zeros((), jnp.float32)
        nlx = jnp.sum(jnp.where(sel1, tx, zero), axis=1, keepdims=True)
        nly = jnp.sum(jnp.where(sel1, ty, zero), axis=1, keepdims=True)
        nlz = jnp.sum(jnp.where(sel1, tz, zero), axis=1, keepdims=True)

        sel = giota == i
        idx_ref[...] = jnp.where(sel, nxt, idx_ref[...])
        cx_ref[...] = jnp.where(sel, nlx, cx_ref[...])
        cy_ref[...] = jnp.where(sel, nly, cy_ref[...])
        cz_ref[...] = jnp.where(sel, nlz, cz_ref[...])
        return nlx, nly, nlz

    jax.lax.fori_loop(1, _G, body, (lx, ly, lz))


def _group_body(xyzp_ref, ctr_ref, chi_ref, cmid_ref, clo_ref,
                gf_ref, nn_ref):
    B = xyzp_ref.shape[0]
    C = xyzp_ref.shape[1]
    for b in range(B):
        pts = xyzp_ref[b]
        cb = ctr_ref[b]
        dots = jax.lax.dot_general(
            pts, cb, (((1,), (0,)), ((), ())),
            preferred_element_type=jnp.float32,
        )
        q2 = jnp.sum(pts * pts, axis=1, keepdims=True)
        k2 = jnp.sum(cb * cb, axis=0, keepdims=True)
        d2 = (q2 + k2) - 2.0 * dots
        giota = jax.lax.broadcasted_iota(jnp.int32, (C, _G), 1)
        nnb = jnp.argmin(d2, axis=1).astype(jnp.int32)[:, None]
        onehot = (giota == nnb).astype(jnp.bfloat16)
        def oh_dot(cref):
            return jax.lax.dot_general(
                onehot, cref[b], (((1,), (0,)), ((), ())),
                preferred_element_type=jnp.float32,
            )
        csel = (oh_dot(chi_ref) + oh_dot(cmid_ref)) + oh_dot(clo_ref)
        nbr = pts - csel
        s = jnp.sum(nbr * nbr, axis=1, keepdims=True)
        dist = jnp.sqrt(s + 1e-16)
        nrm = nbr / jnp.maximum(dist, 1e-8)
        gf_ref[b, :, 0:3] = nrm[:, 0:3]
        gf_ref[b, :, 3:4] = dist
        nn_ref[b, :] = nnb[:, 0]


@jax.jit
def kernel(xyz, features):
    B, N, _ = xyz.shape
    F = features.shape[-1]
    xt = jnp.transpose(xyz, (0, 2, 1))
    x, y, z = xt[:, 0], xt[:, 1], xt[:, 2]

    idx, cx, cy, cz = pl.pallas_call(
        _fps_body,
        out_shape=[
            jax.ShapeDtypeStruct((B, _G), jnp.int32),
            jax.ShapeDtypeStruct((B, _G), jnp.float32),
            jax.ShapeDtypeStruct((B, _G), jnp.float32),
            jax.ShapeDtypeStruct((B, _G), jnp.float32),
        ],
        scratch_shapes=[pltpu.VMEM((B, N), jnp.float32)],
    )(x, y, z)

    centers = jnp.stack([cx, cy, cz], axis=-1)
    xyzp = jnp.pad(xyz, ((0, 0), (0, 0), (0, 5)))
    ctr = jnp.pad(jnp.stack([cx, cy, cz], axis=1),
                  ((0, 0), (0, 5), (0, 0)))
    ctrt = jnp.pad(centers, ((0, 0), (0, 0), (0, 5)))

    def trunc_bf16(v):
        return jax.lax.bitcast_convert_type(
            jax.lax.bitcast_convert_type(v, jnp.uint32) & jnp.uint32(0xFFFF0000),
            jnp.float32)
    chi_f = trunc_bf16(ctrt)
    cmid_f = trunc_bf16(ctrt - chi_f)
    clo_f = (ctrt - chi_f) - cmid_f
    chi = chi_f.astype(jnp.bfloat16)
    cmid = cmid_f.astype(jnp.bfloat16)
    clo = clo_f.astype(jnp.bfloat16)

    CH = 8
    CS = N // CH
    gfa, nn = pl.pallas_call(
        _group_body,
        grid=(CH,),
        in_specs=[
            pl.BlockSpec((B, CS, 8), lambda c: (0, c, 0)),
            pl.BlockSpec((B, 8, _G), lambda c: (0, 0, 0)),
            pl.BlockSpec((B, _G, 8), lambda c: (0, 0, 0)),
            pl.BlockSpec((B, _G, 8), lambda c: (0, 0, 0)),
            pl.BlockSpec((B, _G, 8), lambda c: (0, 0, 0)),
        ],
        out_specs=[
            pl.BlockSpec((B, CS, 4), lambda c: (0, c, 0)),
            pl.BlockSpec((B, CS), lambda c: (0, c)),
        ],
        out_shape=[
            jax.ShapeDtypeStruct((B, N, 4), jnp.float32),
            jax.ShapeDtypeStruct((B, N), jnp.int32),
        ],
    )(xyzp, ctr, chi, cmid, clo)

    gf = jnp.concatenate([gfa, features], axis=-1)
    return gf, centers, nn


# speedup vs baseline: 1.1169x; 1.0060x over previous
import functools

import jax
import jax.numpy as jnp
from jax.experimental import pallas as pl
from jax.experimental.pallas import tpu as pltpu

_G = 512


def _fps_body(x_ref, y_ref, z_ref, idx_ref, cx_ref, cy_ref, cz_ref, dists_ref):
    x = x_ref[...]
    y = y_ref[...]
    z = z_ref[...]
    B, N = x.shape
    iota = jax.lax.broadcasted_iota(jnp.int32, (B, N), 1)

    giota = jax.lax.broadcasted_iota(jnp.int32, (B, _G), 1)

    lx = x[:, 0:1]
    ly = y[:, 0:1]
    lz = z[:, 0:1]
    idx_ref[...] = jnp.zeros((B, _G), jnp.int32)
    cx_ref[...] = jnp.broadcast_to(lx, (B, _G))
    cy_ref[...] = jnp.broadcast_to(ly, (B, _G))
    cz_ref[...] = jnp.broadcast_to(lz, (B, _G))
    dists_ref[...] = jnp.full((B, N), jnp.inf, jnp.float32)

    def body(i, carry):
        lx, ly, lz = carry
        dx = x - lx
        dy = y - ly
        dz = z - lz
        d = (dx * dx + dy * dy) + dz * dz
        dists = jnp.minimum(dists_ref[...], d)
        dists_ref[...] = dists

        t, ti, tx, ty, tz = dists, iota, x, y, z
        w = N
        while w > 128:
            h = w // 2
            a, b = t[:, :h], t[:, h:w]
            cond = a >= b
            t = jnp.where(cond, a, b)
            ti = jnp.where(cond, ti[:, :h], ti[:, h:w])
            tx = jnp.where(cond, tx[:, :h], tx[:, h:w])
            ty = jnp.where(cond, ty[:, :h], ty[:, h:w])
            tz = jnp.where(cond, tz[:, :h], tz[:, h:w])
            w = h
        maxv = jnp.max(t, axis=1, keepdims=True)
        msk = t == maxv
        nxt = jnp.min(jnp.where(msk, ti, jnp.int32(N)), axis=1, keepdims=True)
        sel1 = msk & (ti == nxt)
        zero = jnp.zeros((), jnp.float32)
        nlx = jnp.sum(jnp.where(sel1, tx, zero), axis=1, keepdims=True)
        nly = jnp.sum(jnp.where(sel1, ty, zero), axis=1, keepdims=True)
        nlz = jnp.sum(jnp.where(sel1, tz, zero), axis=1, keepdims=True)

        sel = giota == i
        idx_ref[...] = jnp.where(sel, nxt, idx_ref[...])
        cx_ref[...] = jnp.where(sel, nlx, cx_ref[...])
        cy_ref[...] = jnp.where(sel, nly, cy_ref[...])
        cz_ref[...] = jnp.where(sel, nlz, cz_ref[...])
        return nlx, nly, nlz

    jax.lax.fori_loop(1, _G, body, (lx, ly, lz))


def _group_body(xyzp_ref, ctr_ref, chi_ref, cmid_ref, clo_ref,
                gf_ref, nn_ref):
    B = xyzp_ref.shape[0]
    C = xyzp_ref.shape[1]
    for b in range(B):
        pts = xyzp_ref[b]
        cb = ctr_ref[b]
        dots = jax.lax.dot_general(
            pts, cb, (((1,), (0,)), ((), ())),
            preferred_element_type=jnp.float32,
        )
        q2 = jnp.sum(pts * pts, axis=1, keepdims=True)
        k2 = jnp.sum(cb * cb, axis=0, keepdims=True)
        d2 = (q2 + k2) - 2.0 * dots
        giota = jax.lax.broadcasted_iota(jnp.int32, (C, _G), 1)
        nnb = jnp.argmin(d2, axis=1).astype(jnp.int32)[:, None]
        onehot = (giota == nnb).astype(jnp.bfloat16)
        def oh_dot(cref):
            return jax.lax.dot_general(
                onehot, cref[b], (((1,), (0,)), ((), ())),
                preferred_element_type=jnp.float32,
            )
        csel = (oh_dot(chi_ref) + oh_dot(cmid_ref)) + oh_dot(clo_ref)
        nbr = pts - csel
        s = jnp.sum(nbr * nbr, axis=1, keepdims=True)
        dist = jnp.sqrt(s + 1e-16)
        nrm = nbr / jnp.maximum(dist, 1e-8)
        gf_ref[b, :, 0:3] = nrm[:, 0:3]
        gf_ref[b, :, 3:4] = dist
        nn_ref[b, :] = nnb[:, 0]


@jax.jit
def kernel(xyz, features):
    B, N, _ = xyz.shape
    F = features.shape[-1]
    xt = jnp.transpose(xyz, (0, 2, 1))
    x, y, z = xt[:, 0], xt[:, 1], xt[:, 2]

    idx, cx, cy, cz = pl.pallas_call(
        _fps_body,
        out_shape=[
            jax.ShapeDtypeStruct((B, _G), jnp.int32),
            jax.ShapeDtypeStruct((B, _G), jnp.float32),
            jax.ShapeDtypeStruct((B, _G), jnp.float32),
            jax.ShapeDtypeStruct((B, _G), jnp.float32),
        ],
        scratch_shapes=[pltpu.VMEM((B, N), jnp.float32)],
    )(x, y, z)

    centers = jnp.stack([cx, cy, cz], axis=-1)
    xyzp = jnp.pad(xyz, ((0, 0), (0, 0), (0, 5)))
    ctr = jnp.pad(jnp.stack([cx, cy, cz], axis=1),
                  ((0, 0), (0, 5), (0, 0)))
    ctrt = jnp.pad(centers, ((0, 0), (0, 0), (0, 5)))

    def trunc_bf16(v):
        return jax.lax.bitcast_convert_type(
            jax.lax.bitcast_convert_type(v, jnp.uint32) & jnp.uint32(0xFFFF0000),
            jnp.float32)
    chi_f = trunc_bf16(ctrt)
    cmid_f = trunc_bf16(ctrt - chi_f)
    clo_f = (ctrt - chi_f) - cmid_f
    chi = chi_f.astype(jnp.bfloat16)
    cmid = cmid_f.astype(jnp.bfloat16)
    clo = clo_f.astype(jnp.bfloat16)

    CH = 4
    CS = N // CH
    gfa, nn = pl.pallas_call(
        _group_body,
        grid=(CH,),
        in_specs=[
            pl.BlockSpec((B, CS, 8), lambda c: (0, c, 0)),
            pl.BlockSpec((B, 8, _G), lambda c: (0, 0, 0)),
            pl.BlockSpec((B, _G, 8), lambda c: (0, 0, 0)),
            pl.BlockSpec((B, _G, 8), lambda c: (0, 0, 0)),
            pl.BlockSpec((B, _G, 8), lambda c: (0, 0, 0)),
        ],
        out_specs=[
            pl.BlockSpec((B, CS, 4), lambda c: (0, c, 0)),
            pl.BlockSpec((B, CS), lambda c: (0, c)),
        ],
        out_shape=[
            jax.ShapeDtypeStruct((B, N, 4), jnp.float32),
            jax.ShapeDtypeStruct((B, N), jnp.int32),
        ],
    )(xyzp, ctr, chi, cmid, clo)

    gf = jnp.concatenate([gfa, features], axis=-1)
    return gf, centers, nn
